# baseline (speedup 1.0000x reference)
import functools
from math import sqrt

import jax
import jax.numpy as jnp
from jax import lax
from jax.experimental import pallas as pl
from jax.experimental.pallas import tpu as pltpu
from jax.experimental.pallas import tpu_sc as plsc

D_MODEL = 64
SCALE = float(sqrt(D_MODEL))
LANES = 16

NUM_CORES = 2
NUM_SUBCORES = 16
NUM_WORKERS = NUM_CORES * NUM_SUBCORES

NRING = 4


@functools.lru_cache(maxsize=None)
def _make_lookup(J: int, I: int, D: int):
    assert I % (128 * NUM_WORKERS) == 0 or I == 128 * NUM_WORKERS
    assert J % NRING == 0
    CB = I // NUM_WORKERS
    mesh = plsc.VectorSubcoreMesh(core_axis_name="c", subcore_axis_name="s")

    @functools.partial(
        pl.kernel,
        mesh=mesh,
        out_type=jax.ShapeDtypeStruct((J, D, I), jnp.float32),
        scratch_types=(
            [pltpu.VMEM((J, CB), jnp.int32)]
            + [pltpu.VMEM((CB,), jnp.int32) for _ in range(NRING)]
            + [pltpu.VMEM((CB,), jnp.int32) for _ in range(NRING)]
            + [pltpu.VMEM((CB, 2 * D), jnp.float32) for _ in range(NRING)]
            + [pltpu.VMEM((D, CB), jnp.float32) for _ in range(NRING)]
            + [pltpu.SemaphoreType.DMA for _ in range(2 * NRING)]
        ),
        compiler_params=pltpu.CompilerParams(needs_layout_passes=False),
    )
    def lookup(x_hbm, table_hbm, out_hbm, xcol_v, *bufs):
        pibuf = bufs[0:NRING]
        colbuf = bufs[NRING:2 * NRING]
        gbuf = bufs[2 * NRING:3 * NRING]
        obuf = bufs[3 * NRING:4 * NRING]
        gsem = bufs[4 * NRING:5 * NRING]
        ssem = bufs[5 * NRING:6 * NRING]

        wid = lax.axis_index("s") * NUM_CORES + lax.axis_index("c")
        cbase = wid * CB
        pltpu.sync_copy(x_hbm.at[:, pl.ds(cbase, CB)], xcol_v)

        def prep_and_gather(j, s):
            for g in range(CB // LANES):
                sl = pl.ds(g * LANES, LANES)
                v = xcol_v[j, sl]
                pibuf[s][sl] = lax.shift_right_logical(v, 1)
                colbuf[s][sl] = lax.shift_left(v & 1, 6)
            pltpu.async_copy(table_hbm.at[pibuf[s]], gbuf[s], gsem[s])

        def wait_gather(s):
            pltpu.make_async_copy(table_hbm.at[pibuf[s]], gbuf[s], gsem[s]).wait()

        def start_store(j, s):
            pltpu.async_copy(
                obuf[s], out_hbm.at[j, :, pl.ds(cbase, CB)], ssem[s]
            )

        def wait_store(s):
            pltpu.make_async_copy(
                obuf[s], out_hbm.at[0, :, pl.ds(cbase, CB)], ssem[s]
            ).wait()

        def transform(s):
            gb, ob, cb = gbuf[s], obuf[s], colbuf[s]
            for g in range(CB // LANES):
                sl = pl.ds(g * LANES, LANES)
                rowv = lax.iota(jnp.int32, LANES) + (g * LANES)
                colb = cb[sl]

                @plsc.parallel_loop(0, D, 1, unroll=4)
                def _(d):
                    v = plsc.load_gather(gb, [rowv, colb + d])
                    ob[d, sl] = v * SCALE

        for s in range(NRING):
            prep_and_gather(s, s)

        def outer(o, carry):
            for s in range(NRING):
                j = o * NRING + s
                wait_gather(s)

                @pl.when(j >= NRING)
                def _():
                    wait_store(s)

                transform(s)
                start_store(j, s)
                jn = j + NRING

                @pl.when(jn < J)
                def _():
                    prep_and_gather(jn, s)

            return carry

        lax.fori_loop(0, J // NRING, outer, 0)

        for s in range(NRING):
            wait_store(s)

    return lookup


def kernel(x, table):
    J, I = x.shape[1], x.shape[0]
    xT = x.T.astype(jnp.int32)
    table2 = table.reshape(table.shape[0] // 2, 2 * D_MODEL)
    outP = _make_lookup(J, I, D_MODEL)(xT, table2)
    return outP.transpose(2, 0, 1)


# speedup vs baseline: 1.0423x; 1.0423x over previous
import functools
from math import sqrt

import jax
import jax.numpy as jnp
from jax import lax
from jax.experimental import pallas as pl
from jax.experimental.pallas import tpu as pltpu
from jax.experimental.pallas import tpu_sc as plsc

D_MODEL = 64
SCALE = float(sqrt(D_MODEL))
LANES = 16

NUM_CORES = 2
NUM_SUBCORES = 16
NUM_WORKERS = NUM_CORES * NUM_SUBCORES

NRING = 4
SZ = (104, 96)
OFF = (0, 104)


@functools.lru_cache(maxsize=None)
def _make_lookup(NI: int, NJ: int, D: int):
    IPW = NI // NUM_WORKERS
    NH = 2 * IPW
    assert NH % NRING == 0
    mesh = plsc.VectorSubcoreMesh(core_axis_name="c", subcore_axis_name="s")

    @functools.partial(
        pl.kernel,
        mesh=mesh,
        out_type=jax.ShapeDtypeStruct((NI, NJ, D), jnp.float32),
        scratch_types=(
            [pltpu.VMEM((IPW, NJ), jnp.int32)]
            + [pltpu.VMEM((SZ[s % 2], 2 * D), jnp.float32) for s in range(NRING)]
            + [pltpu.SemaphoreType.DMA for _ in range(2 * NRING)]
        ),
        compiler_params=pltpu.CompilerParams(
            use_tc_tiling_on_sc=False, needs_layout_passes=False
        ),
    )
    def lookup(x_hbm, tab_hbm, out_hbm, xidx_v, *rest):
        gbuf = rest[0:NRING]
        gsem = rest[NRING:2 * NRING]
        ssem = rest[2 * NRING:3 * NRING]

        wid = lax.axis_index("s") * NUM_CORES + lax.axis_index("c")
        ibase = wid * IPW
        pltpu.sync_copy(x_hbm.at[pl.ds(ibase, IPW), :], xidx_v)

        def gather(h, s):
            il = lax.shift_right_logical(h, 1)
            off, sz = OFF[s % 2], SZ[s % 2]
            return pltpu.make_async_copy(
                tab_hbm.at[xidx_v.at[il, pl.ds(off, sz)]], gbuf[s], gsem[s]
            )

        def store(h, s):
            il = lax.shift_right_logical(h, 1)
            off, sz = OFF[s % 2], SZ[s % 2]
            return pltpu.make_async_copy(
                gbuf[s].at[:, pl.ds(0, D)],
                out_hbm.at[ibase + il, pl.ds(off, sz), :],
                ssem[s],
            )

        def scale(s):
            gb, sz = gbuf[s], SZ[s % 2]

            @plsc.parallel_loop(0, sz, 1, unroll=4)
            def _(r):
                for k in range(D // LANES):
                    sl = pl.ds(k * LANES, LANES)
                    gb[r, sl] = gb[r, sl] * SCALE

        for s in range(NRING - 1):
            gather(s, s).start()

        def outer(o, carry):
            for s in range(NRING):
                h = o * NRING + s
                gather(h, s).wait()
                scale(s)
                store(h, s).start()
                sp = (s - 1) % NRING
                hn = h + NRING - 1

                @pl.when(hn < NH)
                def _():
                    @pl.when(h > 0)
                    def _():
                        store(h - 1, sp).wait()

                    gather(hn, sp).start()

            return carry

        lax.fori_loop(0, NH // NRING, outer, 0)

        for s in range(NRING):
            store(NH - NRING + s, s).wait()

    return lookup


def kernel(x, table):
    NI, NJ = x.shape
    xi = x.astype(jnp.int32)
    tab_p = jnp.pad(table, ((0, 0), (0, D_MODEL)))
    return _make_lookup(NI, NJ, D_MODEL)(xi, tab_p)
